# baseline (speedup 1.0000x reference)
import functools

import jax
import jax.numpy as jnp
from jax import lax
from jax.experimental import pallas as pl
from jax.experimental.pallas import tpu as pltpu
from jax.experimental.pallas import tpu_sc as plsc

NN = 10000
NNP = 10240
EE = 160000
NC = 2
NS = 16
EPT = EE // NS
ROWS_PT = NNP // NS
KB_D = 200
DEG_EPT = EE // (NC * NS)
NB_D = DEG_EPT // KB_D
FC = 160

_BM = 400
_GRID = NN // _BM


def _deg_body(dst_hbm, ones_hbm, zeros_hbm, out_hbm, idxd, ones_v, sems, acc):
    c = lax.axis_index("c")
    s = lax.axis_index("s")
    r0 = s * ROWS_PT
    t = c * NS + s
    pltpu.sync_copy(zeros_hbm.at[pl.ds(r0, ROWS_PT)], acc.at[pl.ds(r0, ROWS_PT)])
    pltpu.sync_copy(dst_hbm.at[pl.ds(t * NB_D, NB_D)], idxd)
    pltpu.sync_copy(ones_hbm, ones_v)
    plsc.subcore_barrier()

    def fire(b, carry):
        pltpu.async_copy(ones_v, acc.at[idxd.at[b]], sems, add=True)
        return carry

    lax.fori_loop(0, NB_D, fire, 0)

    def drain(b, carry):
        pltpu.make_async_copy(ones_v, acc.at[idxd.at[0]], sems).wait()
        return carry

    lax.fori_loop(0, NB_D, drain, 0)
    plsc.subcore_barrier()
    pltpu.sync_copy(acc.at[pl.ds(r0, ROWS_PT)],
                    out_hbm.at[pl.ds(c * NNP + r0, ROWS_PT)])


@functools.cache
def _deg_call():
    mesh = plsc.VectorSubcoreMesh(core_axis_name="c", subcore_axis_name="s")
    return pl.kernel(
        _deg_body,
        out_type=jax.ShapeDtypeStruct((NC * NNP, 16), jnp.float32),
        mesh=mesh,
        compiler_params=pltpu.CompilerParams(use_tc_tiling_on_sc=False),
        scratch_types=[
            pltpu.VMEM((NB_D, KB_D), jnp.int32),
            pltpu.VMEM((KB_D, 16), jnp.float32),
            pltpu.SemaphoreType.DMA,
            pltpu.VMEM_SHARED((NNP, 16), jnp.float32),
        ],
    )


def _prop_f_body(ngroups, d2, kb, leaky,
                 src_hbm, dst_hbm, z_hbm, init_hbm, scale_hbm, out_hbm,
                 idxs, idxd, rows0, rows1, scale_v,
                 semg0, semg1, sems0, sems1, acc):
    nb = EPT // kb
    c = lax.axis_index("c")
    s = lax.axis_index("s")
    r0 = s * ROWS_PT
    pltpu.sync_copy(dst_hbm.at[pl.ds(s * nb, nb)], idxd)
    pltpu.sync_copy(src_hbm.at[pl.ds(s * nb, nb)], idxs)
    pltpu.sync_copy(scale_hbm.at[pl.ds(r0, ROWS_PT)], scale_v)
    for g in range(ngroups):
        j = g * NC + c
        pltpu.sync_copy(init_hbm.at[j, pl.ds(r0, ROWS_PT)],
                        acc.at[pl.ds(r0, ROWS_PT)])
        plsc.subcore_barrier()
        pltpu.async_copy(z_hbm.at[j].at[idxs.at[0]], rows0, semg0)

        def body(i, carry):
            b0 = 2 * i
            b1 = 2 * i + 1
            pltpu.make_async_copy(z_hbm.at[j].at[idxs.at[b0]], rows0, semg0).wait()

            @pl.when(i > 0)
            def _():
                pltpu.make_async_copy(rows1, acc.at[idxd.at[0]], sems1).wait()

            pltpu.async_copy(z_hbm.at[j].at[idxs.at[b1]], rows1, semg1)
            pltpu.async_copy(rows0, acc.at[idxd.at[b0]], sems0, add=True)
            pltpu.make_async_copy(z_hbm.at[j].at[idxs.at[b1]], rows1, semg1).wait()

            @pl.when(i < nb // 2 - 1)
            def _():
                pltpu.make_async_copy(rows0, acc.at[idxd.at[0]], sems0).wait()
                pltpu.async_copy(z_hbm.at[j].at[idxs.at[b0 + 2]], rows0, semg0)

            pltpu.async_copy(rows1, acc.at[idxd.at[b1]], sems1, add=True)
            return carry

        lax.fori_loop(0, nb // 2, body, 0)
        pltpu.make_async_copy(rows0, acc.at[idxd.at[0]], sems0).wait()
        pltpu.make_async_copy(rows1, acc.at[idxd.at[0]], sems1).wait()
        plsc.subcore_barrier()
        for m in range(ROWS_PT // FC):
            pltpu.sync_copy(acc.at[pl.ds(r0 + m * FC, FC)],
                            rows0.at[pl.ds(0, FC)])

            def srow(r, carry):
                idxv = jnp.full((16,), m * FC + r, jnp.int32)
                sc = plsc.load_gather(scale_v, [idxv])
                for jj in range(d2 // 16):
                    vec = rows0[r, pl.ds(jj * 16, 16)] * sc
                    if leaky:
                        vec = jnp.where(vec >= 0, vec, 0.01 * vec)
                    rows0[r, pl.ds(jj * 16, 16)] = vec
                return carry

            lax.fori_loop(0, FC, srow, 0)
            pltpu.sync_copy(rows0.at[pl.ds(0, FC)],
                            out_hbm.at[j, pl.ds(r0 + m * FC, FC)])


@functools.cache
def _prop_f_call(d2, ngroups, leaky):
    kb = 1000 if d2 == 32 else 200
    nb = EPT // kb
    mesh = plsc.VectorSubcoreMesh(core_axis_name="c", subcore_axis_name="s")
    return pl.kernel(
        functools.partial(_prop_f_body, ngroups, d2, kb, leaky),
        out_type=jax.ShapeDtypeStruct((ngroups * NC, NNP, d2), jnp.float32),
        mesh=mesh,
        compiler_params=pltpu.CompilerParams(use_tc_tiling_on_sc=False,
                                             needs_layout_passes=False),
        scratch_types=[
            pltpu.VMEM((nb, kb), jnp.int32),
            pltpu.VMEM((nb, kb), jnp.int32),
            pltpu.VMEM((kb, d2), jnp.float32),
            pltpu.VMEM((kb, d2), jnp.float32),
            pltpu.VMEM((ROWS_PT,), jnp.float32),
            pltpu.SemaphoreType.DMA,
            pltpu.SemaphoreType.DMA,
            pltpu.SemaphoreType.DMA,
            pltpu.SemaphoreType.DMA,
            pltpu.VMEM_SHARED((NNP, d2), jnp.float32),
        ],
    )


def _alpha(ek, v):
    logits = jnp.sum(ek * v, axis=1, keepdims=True)
    m = jnp.max(logits)
    e = jnp.exp(logits - m)
    return e / jnp.sum(e)


def _mm_body(pspec, dout, d2, *refs):
    nch = dout // d2
    nparts = len(pspec)
    parts = refs[:nparts]
    ws = refs[nparts:2 * nparts]
    dinv_r, ek_r, v_r, b_r = refs[2 * nparts:2 * nparts + 4]
    z0m_r, z1m_r, s2_r = refs[2 * nparts + 4:]
    acc = None
    for p, w, kind in zip(parts, ws, pspec):
        pv = p[...]
        if kind[0] == '3d':
            pv = pv[0]
        d = jnp.dot(pv.astype(jnp.bfloat16), w[...].astype(jnp.bfloat16),
                    preferred_element_type=jnp.float32)
        acc = d if acc is None else acc + d
    al = _alpha(ek_r[...], v_r[...])
    dv = dinv_r[...]
    idv = 1.0 / dv
    z0m = acc[:, :dout] * (al[0:1, :] * idv) + idv * b_r[...]
    z1m = acc[:, dout:2 * dout] * (al[1:2, :] * idv)
    s2 = acc[:, 2 * dout:] * (al[2:3, :] * dv)
    for j in range(nch):
        z0m_r[j] = z0m[:, j * d2:(j + 1) * d2]
        z1m_r[j] = z1m[:, j * d2:(j + 1) * d2]
        s2_r[j] = s2[:, j * d2:(j + 1) * d2]


@functools.cache
def _mm_call(pspec, dout, d2):
    nch = dout // d2
    in_specs = []
    for kind in pspec:
        if kind[0] == '2d':
            in_specs.append(pl.BlockSpec((_BM, kind[1]), lambda i: (i, 0)))
        else:
            jj = kind[2]
            in_specs.append(pl.BlockSpec((1, _BM, kind[1]),
                                         lambda i, jj=jj: (jj, i, 0)))
    for kind in pspec:
        in_specs.append(pl.BlockSpec((kind[1], 3 * dout), lambda i: (0, 0)))
    in_specs += [pl.BlockSpec((_BM, 1), lambda i: (i, 0)),
                 pl.BlockSpec((3, 16), lambda i: (0, 0)),
                 pl.BlockSpec((1, 16), lambda i: (0, 0)),
                 pl.BlockSpec((1, dout), lambda i: (0, 0))]
    spec_np = pl.BlockSpec((nch, _BM, d2), lambda i: (0, i, 0))
    out_specs = (spec_np, spec_np, spec_np)
    out_shape = (
        jax.ShapeDtypeStruct((nch, NNP, d2), jnp.float32),
        jax.ShapeDtypeStruct((nch, NNP, d2), jnp.float32),
        jax.ShapeDtypeStruct((nch, NN, d2), jnp.float32),
    )
    return pl.pallas_call(
        functools.partial(_mm_body, pspec, dout, d2),
        grid=(_GRID,),
        in_specs=in_specs,
        out_specs=out_specs,
        out_shape=out_shape,
    )


def _dinv_body(da_r, dinv_r, dvsq_r):
    da = da_r[...]
    deg = da[:NNP, 0:1] + da[NNP:, 0:1]
    dv = lax.rsqrt(jnp.maximum(deg, 1.0))
    dinv_r[...] = dv
    dvsq_r[...] = dv * dv


@functools.cache
def _dinv_call():
    return pl.pallas_call(
        _dinv_body,
        out_shape=(jax.ShapeDtypeStruct((NNP, 1), jnp.float32),
                   jax.ShapeDtypeStruct((NNP, 1), jnp.float32)),
    )


def _layer(parts, wcat, b, ek, v, srcs, dinv, dv1, dv2, dout, d2):
    nch = dout // d2
    ngroups = nch // NC
    kb = 1000 if d2 == 32 else 200
    pspec = []
    ops = []
    wparts = []
    off = 0
    for f in parts:
        if f.ndim == 2:
            dp = f.shape[1]
            pspec.append(('2d', dp))
            ops.append(f)
            wparts.append(wcat[off:off + dp])
            off += dp
        else:
            nchp, _, d2p = f.shape
            for j in range(nchp):
                pspec.append(('3d', d2p, j))
                ops.append(f)
                wparts.append(wcat[off:off + d2p])
                off += d2p
    v2 = v.reshape(1, -1)
    z0m, z1m, s2 = _mm_call(tuple(pspec), dout, d2)(
        *ops, *wparts, dinv, ek, v2, b.reshape(1, -1))
    src2d, dst2d = srcs[kb]
    s1c = _prop_f_call(d2, ngroups, False)(src2d, dst2d, s2, z1m, dv2)
    h = _prop_f_call(d2, ngroups, True)(src2d, dst2d, s1c, z0m, dv1)
    return h


def kernel(x, edge_index, W1, b1, Ek1, v1, W2, b2, Ek2, v2,
           W3, b3, Ek3, v3, W4, b4, Ek4, v4):
    src = edge_index[0]
    dst = edge_index[1]
    srcs = {
        200: (src.reshape(NS * 50, 200), dst.reshape(NS * 50, 200)),
        1000: (src.reshape(NS * 10, 1000), dst.reshape(NS * 10, 1000)),
    }

    ones = jnp.ones((KB_D, 16), jnp.float32)
    zeros16 = jnp.zeros((NNP, 16), jnp.float32)
    deg_acc = _deg_call()(dst.reshape(NS * 50, KB_D), ones, zeros16)
    dinv, dvsq = _dinv_call()(deg_acc)
    dv1 = dinv.reshape(NNP)
    dv2 = dvsq.reshape(NNP)

    W4p = jnp.pad(W4, ((0, 0), (0, 0), (0, 24)))
    b4p = jnp.pad(b4, (0, 24))

    def wcat(W):
        return jnp.concatenate([W[0], W[1], W[2]], axis=1)

    h1 = _layer([x], wcat(W1), b1, Ek1, v1, srcs, dinv, dv1, dv2, 256, 64)
    h2 = _layer([h1, x], wcat(W2), b2, Ek2, v2, srcs, dinv, dv1, dv2, 128, 64)
    h3 = _layer([h2, h1, x], wcat(W3), b3, Ek3, v3, srcs, dinv, dv1, dv2,
                64, 32)
    h4 = _layer([h3, h2, h1, x], wcat(W4p), b4p, Ek4, v4, srcs, dinv, dv1, dv2,
                64, 32)
    return jnp.concatenate([h4[0, :NN], h4[1, :NN]], axis=1)[:, :40]


# speedup vs baseline: 1.0747x; 1.0747x over previous
import functools

import jax
import jax.numpy as jnp
from jax import lax
from jax.experimental import pallas as pl
from jax.experimental.pallas import tpu as pltpu
from jax.experimental.pallas import tpu_sc as plsc

NN = 10000
NNP = 10240
EE = 160000
NC = 2
NS = 16
EPT = EE // NS
ROWS_PT = NNP // NS
KB_D = 200
DEG_EPT = EE // (NC * NS)
NB_D = DEG_EPT // KB_D
FC = 160

_BM = 400
_GRID = NN // _BM


def _deg_body(dst_hbm, ones_hbm, zeros_hbm, out_hbm, idxd, ones_v, sems, acc):
    c = lax.axis_index("c")
    s = lax.axis_index("s")
    r0 = s * ROWS_PT
    t = c * NS + s
    pltpu.sync_copy(zeros_hbm.at[pl.ds(r0, ROWS_PT)], acc.at[pl.ds(r0, ROWS_PT)])
    pltpu.sync_copy(dst_hbm.at[pl.ds(t * NB_D, NB_D)], idxd)
    pltpu.sync_copy(ones_hbm, ones_v)
    plsc.subcore_barrier()

    def fire(b, carry):
        pltpu.async_copy(ones_v, acc.at[idxd.at[b]], sems, add=True)
        return carry

    lax.fori_loop(0, NB_D, fire, 0)

    def drain(b, carry):
        pltpu.make_async_copy(ones_v, acc.at[idxd.at[0]], sems).wait()
        return carry

    lax.fori_loop(0, NB_D, drain, 0)
    plsc.subcore_barrier()
    pltpu.sync_copy(acc.at[pl.ds(r0, ROWS_PT)],
                    out_hbm.at[pl.ds(c * NNP + r0, ROWS_PT)])


@functools.cache
def _deg_call():
    mesh = plsc.VectorSubcoreMesh(core_axis_name="c", subcore_axis_name="s")
    return pl.kernel(
        _deg_body,
        out_type=jax.ShapeDtypeStruct((NC * NNP, 16), jnp.float32),
        mesh=mesh,
        compiler_params=pltpu.CompilerParams(use_tc_tiling_on_sc=False),
        scratch_types=[
            pltpu.VMEM((NB_D, KB_D), jnp.int32),
            pltpu.VMEM((KB_D, 16), jnp.float32),
            pltpu.SemaphoreType.DMA,
            pltpu.VMEM_SHARED((NNP, 16), jnp.float32),
        ],
    )


def _prop_f_body(ngroups, d2, kb, leaky,
                 src_hbm, dst_hbm, z_hbm, init_hbm, scale_hbm, out_hbm,
                 idxs, idxd, rows0, rows1, scale_v,
                 semg0, semg1, sems0, sems1, acc):
    nb = EPT // kb
    c = lax.axis_index("c")
    s = lax.axis_index("s")
    r0 = s * ROWS_PT
    pltpu.sync_copy(dst_hbm.at[pl.ds(s * nb, nb)], idxd)
    pltpu.sync_copy(src_hbm.at[pl.ds(s * nb, nb)], idxs)
    pltpu.sync_copy(scale_hbm.at[pl.ds(r0, ROWS_PT)], scale_v)
    for g in range(ngroups):
        j = g * NC + c
        pltpu.sync_copy(init_hbm.at[j, pl.ds(r0, ROWS_PT)],
                        acc.at[pl.ds(r0, ROWS_PT)])
        plsc.subcore_barrier()
        pltpu.async_copy(z_hbm.at[j].at[idxs.at[0]], rows0, semg0)

        def body(i, carry):
            b0 = 2 * i
            b1 = 2 * i + 1
            pltpu.make_async_copy(z_hbm.at[j].at[idxs.at[b0]], rows0, semg0).wait()

            @pl.when(i > 0)
            def _():
                pltpu.make_async_copy(rows1, acc.at[idxd.at[0]], sems1).wait()

            pltpu.async_copy(z_hbm.at[j].at[idxs.at[b1]], rows1, semg1)
            pltpu.async_copy(rows0, acc.at[idxd.at[b0]], sems0, add=True)
            pltpu.make_async_copy(z_hbm.at[j].at[idxs.at[b1]], rows1, semg1).wait()

            lim = nb // 2 - 1 if nb % 2 == 0 else nb // 2

            @pl.when(i < lim)
            def _():
                pltpu.make_async_copy(rows0, acc.at[idxd.at[0]], sems0).wait()
                pltpu.async_copy(z_hbm.at[j].at[idxs.at[b0 + 2]], rows0, semg0)

            pltpu.async_copy(rows1, acc.at[idxd.at[b1]], sems1, add=True)
            return carry

        lax.fori_loop(0, nb // 2, body, 0)
        if nb % 2 == 1:
            bl = nb - 1
            pltpu.make_async_copy(z_hbm.at[j].at[idxs.at[bl]], rows0,
                                  semg0).wait()
            pltpu.make_async_copy(rows1, acc.at[idxd.at[0]], sems1).wait()
            pltpu.async_copy(rows0, acc.at[idxd.at[bl]], sems0, add=True)
            pltpu.make_async_copy(rows0, acc.at[idxd.at[0]], sems0).wait()
        else:
            pltpu.make_async_copy(rows0, acc.at[idxd.at[0]], sems0).wait()
            pltpu.make_async_copy(rows1, acc.at[idxd.at[0]], sems1).wait()
        plsc.subcore_barrier()
        for m in range(ROWS_PT // FC):
            pltpu.sync_copy(acc.at[pl.ds(r0 + m * FC, FC)],
                            rows0.at[pl.ds(0, FC)])

            def srow(r, carry):
                idxv = jnp.full((16,), m * FC + r, jnp.int32)
                sc = plsc.load_gather(scale_v, [idxv])
                for jj in range(d2 // 16):
                    vec = rows0[r, pl.ds(jj * 16, 16)] * sc
                    if leaky:
                        vec = jnp.where(vec >= 0, vec, 0.01 * vec)
                    rows0[r, pl.ds(jj * 16, 16)] = vec
                return carry

            lax.fori_loop(0, FC, srow, 0)
            pltpu.sync_copy(rows0.at[pl.ds(0, FC)],
                            out_hbm.at[j, pl.ds(r0 + m * FC, FC)])


@functools.cache
def _prop_f_call(d2, ngroups, leaky):
    kb = 1000 if d2 == 32 else 400
    nb = EPT // kb
    mesh = plsc.VectorSubcoreMesh(core_axis_name="c", subcore_axis_name="s")
    return pl.kernel(
        functools.partial(_prop_f_body, ngroups, d2, kb, leaky),
        out_type=jax.ShapeDtypeStruct((ngroups * NC, NNP, d2), jnp.float32),
        mesh=mesh,
        compiler_params=pltpu.CompilerParams(use_tc_tiling_on_sc=False,
                                             needs_layout_passes=False),
        scratch_types=[
            pltpu.VMEM((nb, kb), jnp.int32),
            pltpu.VMEM((nb, kb), jnp.int32),
            pltpu.VMEM((kb, d2), jnp.float32),
            pltpu.VMEM((kb, d2), jnp.float32),
            pltpu.VMEM((ROWS_PT,), jnp.float32),
            pltpu.SemaphoreType.DMA,
            pltpu.SemaphoreType.DMA,
            pltpu.SemaphoreType.DMA,
            pltpu.SemaphoreType.DMA,
            pltpu.VMEM_SHARED((NNP, d2), jnp.float32),
        ],
    )


def _alpha(ek, v):
    logits = jnp.sum(ek * v, axis=1, keepdims=True)
    m = jnp.max(logits)
    e = jnp.exp(logits - m)
    return e / jnp.sum(e)


def _mm_body(pspec, dout, d2, *refs):
    nch = dout // d2
    nparts = len(pspec)
    parts = refs[:nparts]
    ws = refs[nparts:2 * nparts]
    dinv_r, ek_r, v_r, b_r = refs[2 * nparts:2 * nparts + 4]
    z0m_r, z1m_r, s2_r = refs[2 * nparts + 4:]
    acc = None
    for p, w, kind in zip(parts, ws, pspec):
        pv = p[...]
        if kind[0] == '3d':
            pv = pv[0]
        d = jnp.dot(pv, w[...], preferred_element_type=jnp.float32)
        acc = d if acc is None else acc + d
    al = _alpha(ek_r[...], v_r[...])
    dv = dinv_r[...]
    idv = 1.0 / dv
    z0m = acc[:, :dout] * (al[0:1, :] * idv) + idv * b_r[...]
    z1m = acc[:, dout:2 * dout] * (al[1:2, :] * idv)
    s2 = acc[:, 2 * dout:] * (al[2:3, :] * dv)
    for j in range(nch):
        z0m_r[j] = z0m[:, j * d2:(j + 1) * d2]
        z1m_r[j] = z1m[:, j * d2:(j + 1) * d2]
        s2_r[j] = s2[:, j * d2:(j + 1) * d2]


@functools.cache
def _mm_call(pspec, dout, d2):
    nch = dout // d2
    in_specs = []
    for kind in pspec:
        if kind[0] == '2d':
            in_specs.append(pl.BlockSpec((_BM, kind[1]), lambda i: (i, 0)))
        else:
            jj = kind[2]
            in_specs.append(pl.BlockSpec((1, _BM, kind[1]),
                                         lambda i, jj=jj: (jj, i, 0)))
    for kind in pspec:
        in_specs.append(pl.BlockSpec((kind[1], 3 * dout), lambda i: (0, 0)))
    in_specs += [pl.BlockSpec((_BM, 1), lambda i: (i, 0)),
                 pl.BlockSpec((3, 16), lambda i: (0, 0)),
                 pl.BlockSpec((1, 16), lambda i: (0, 0)),
                 pl.BlockSpec((1, dout), lambda i: (0, 0))]
    spec_np = pl.BlockSpec((nch, _BM, d2), lambda i: (0, i, 0))
    out_specs = (spec_np, spec_np, spec_np)
    out_shape = (
        jax.ShapeDtypeStruct((nch, NNP, d2), jnp.float32),
        jax.ShapeDtypeStruct((nch, NNP, d2), jnp.float32),
        jax.ShapeDtypeStruct((nch, NN, d2), jnp.float32),
    )
    return pl.pallas_call(
        functools.partial(_mm_body, pspec, dout, d2),
        grid=(_GRID,),
        in_specs=in_specs,
        out_specs=out_specs,
        out_shape=out_shape,
    )


def _dinv_body(da_r, dinv_r, dvsq_r):
    da = da_r[...]
    deg = da[:NNP, 0:1] + da[NNP:, 0:1]
    dv = lax.rsqrt(jnp.maximum(deg, 1.0))
    dinv_r[...] = dv
    dvsq_r[...] = dv * dv


@functools.cache
def _dinv_call():
    return pl.pallas_call(
        _dinv_body,
        out_shape=(jax.ShapeDtypeStruct((NNP, 1), jnp.float32),
                   jax.ShapeDtypeStruct((NNP, 1), jnp.float32)),
    )


def _layer(parts, wcat, b, ek, v, srcs, dinv, dv1, dv2, dout, d2):
    nch = dout // d2
    ngroups = nch // NC
    kb = 1000 if d2 == 32 else 400
    pspec = []
    ops = []
    wparts = []
    off = 0
    for f in parts:
        if f.ndim == 2:
            dp = f.shape[1]
            pspec.append(('2d', dp))
            ops.append(f)
            wparts.append(wcat[off:off + dp])
            off += dp
        else:
            nchp, _, d2p = f.shape
            for j in range(nchp):
                pspec.append(('3d', d2p, j))
                ops.append(f)
                wparts.append(wcat[off:off + d2p])
                off += d2p
    v2 = v.reshape(1, -1)
    z0m, z1m, s2 = _mm_call(tuple(pspec), dout, d2)(
        *ops, *wparts, dinv, ek, v2, b.reshape(1, -1))
    src2d, dst2d = srcs[kb]
    s1c = _prop_f_call(d2, ngroups, False)(src2d, dst2d, s2, z1m, dv2)
    h = _prop_f_call(d2, ngroups, True)(src2d, dst2d, s1c, z0m, dv1)
    return h


def kernel(x, edge_index, W1, b1, Ek1, v1, W2, b2, Ek2, v2,
           W3, b3, Ek3, v3, W4, b4, Ek4, v4):
    src = edge_index[0]
    dst = edge_index[1]
    srcs = {
        400: (src.reshape(NS * 25, 400), dst.reshape(NS * 25, 400)),
        1000: (src.reshape(NS * 10, 1000), dst.reshape(NS * 10, 1000)),
    }

    ones = jnp.ones((KB_D, 16), jnp.float32)
    zeros16 = jnp.zeros((NNP, 16), jnp.float32)
    deg_acc = _deg_call()(dst.reshape(NS * 50, KB_D), ones, zeros16)
    dinv, dvsq = _dinv_call()(deg_acc)
    dv1 = dinv.reshape(NNP)
    dv2 = dvsq.reshape(NNP)

    W4p = jnp.pad(W4, ((0, 0), (0, 0), (0, 24)))
    b4p = jnp.pad(b4, (0, 24))

    def wcat(W):
        return jnp.concatenate([W[0], W[1], W[2]], axis=1)

    h1 = _layer([x], wcat(W1), b1, Ek1, v1, srcs, dinv, dv1, dv2, 256, 64)
    h2 = _layer([h1, x], wcat(W2), b2, Ek2, v2, srcs, dinv, dv1, dv2, 128, 64)
    h3 = _layer([h2, h1, x], wcat(W3), b3, Ek3, v3, srcs, dinv, dv1, dv2,
                64, 32)
    h4 = _layer([h3, h2, h1, x], wcat(W4p), b4p, Ek4, v4, srcs, dinv, dv1, dv2,
                64, 32)
    return jnp.concatenate([h4[0, :NN], h4[1, :NN]], axis=1)[:, :40]


# speedup vs baseline: 1.1075x; 1.0305x over previous
import functools

import jax
import jax.numpy as jnp
from jax import lax
from jax.experimental import pallas as pl
from jax.experimental.pallas import tpu as pltpu
from jax.experimental.pallas import tpu_sc as plsc

NN = 10000
NNP = 10240
EE = 160000
NC = 2
NS = 16
EPT = EE // NS
ROWS_PT = NNP // NS
KB_D = 200
DEG_EPT = EE // (NC * NS)
NB_D = DEG_EPT // KB_D
FC = 160

_BM = 400
_GRID = NN // _BM


def _deg_body(dst_hbm, ones_hbm, zeros_hbm, out_hbm, idxd, ones_v, sems, acc):
    c = lax.axis_index("c")
    s = lax.axis_index("s")
    r0 = s * ROWS_PT
    t = c * NS + s
    pltpu.sync_copy(zeros_hbm.at[pl.ds(r0, ROWS_PT)], acc.at[pl.ds(r0, ROWS_PT)])
    pltpu.sync_copy(dst_hbm.at[pl.ds(t * NB_D, NB_D)], idxd)
    pltpu.sync_copy(ones_hbm, ones_v)
    plsc.subcore_barrier()

    def fire(b, carry):
        pltpu.async_copy(ones_v, acc.at[idxd.at[b]], sems, add=True)
        return carry

    lax.fori_loop(0, NB_D, fire, 0)

    def drain(b, carry):
        pltpu.make_async_copy(ones_v, acc.at[idxd.at[0]], sems).wait()
        return carry

    lax.fori_loop(0, NB_D, drain, 0)
    plsc.subcore_barrier()
    pltpu.sync_copy(acc.at[pl.ds(r0, ROWS_PT)],
                    out_hbm.at[pl.ds(c * NNP + r0, ROWS_PT)])


@functools.cache
def _deg_call():
    mesh = plsc.VectorSubcoreMesh(core_axis_name="c", subcore_axis_name="s")
    return pl.kernel(
        _deg_body,
        out_type=jax.ShapeDtypeStruct((NC * NNP, 16), jnp.float32),
        mesh=mesh,
        compiler_params=pltpu.CompilerParams(use_tc_tiling_on_sc=False),
        scratch_types=[
            pltpu.VMEM((NB_D, KB_D), jnp.int32),
            pltpu.VMEM((KB_D, 16), jnp.float32),
            pltpu.SemaphoreType.DMA,
            pltpu.VMEM_SHARED((NNP, 16), jnp.float32),
        ],
    )


def _prop_f_body(ngroups, d2, kb, leaky,
                 src_hbm, dst_hbm, z_hbm, init_hbm, scale_hbm, out_hbm,
                 idxs, idxd, rows0, rows1, scale_v,
                 semg0, semg1, sems0, sems1, acc):
    nb = EPT // kb
    c = lax.axis_index("c")
    s = lax.axis_index("s")
    r0 = s * ROWS_PT
    pd1 = pltpu.async_copy(dst_hbm.at[pl.ds(s * nb, nb)], idxd, semg0)
    pd2 = pltpu.async_copy(src_hbm.at[pl.ds(s * nb, nb)], idxs, semg1)
    pd3 = pltpu.async_copy(scale_hbm.at[pl.ds(r0, ROWS_PT)], scale_v, sems0)
    pd4 = pltpu.async_copy(init_hbm.at[c, pl.ds(r0, ROWS_PT)],
                           acc.at[pl.ds(r0, ROWS_PT)], sems1)
    pd1.wait()
    pd2.wait()
    pd3.wait()
    pd4.wait()
    for g in range(ngroups):
        j = g * NC + c
        if g > 0:
            pltpu.sync_copy(init_hbm.at[j, pl.ds(r0, ROWS_PT)],
                            acc.at[pl.ds(r0, ROWS_PT)])
        plsc.subcore_barrier()
        pltpu.async_copy(z_hbm.at[j].at[idxs.at[0]], rows0, semg0)

        def body(i, carry):
            b0 = 2 * i
            b1 = 2 * i + 1
            pltpu.make_async_copy(z_hbm.at[j].at[idxs.at[b0]], rows0, semg0).wait()

            @pl.when(i > 0)
            def _():
                pltpu.make_async_copy(rows1, acc.at[idxd.at[0]], sems1).wait()

            pltpu.async_copy(z_hbm.at[j].at[idxs.at[b1]], rows1, semg1)
            pltpu.async_copy(rows0, acc.at[idxd.at[b0]], sems0, add=True)
            pltpu.make_async_copy(z_hbm.at[j].at[idxs.at[b1]], rows1, semg1).wait()

            lim = nb // 2 - 1 if nb % 2 == 0 else nb // 2

            @pl.when(i < lim)
            def _():
                pltpu.make_async_copy(rows0, acc.at[idxd.at[0]], sems0).wait()
                pltpu.async_copy(z_hbm.at[j].at[idxs.at[b0 + 2]], rows0, semg0)

            pltpu.async_copy(rows1, acc.at[idxd.at[b1]], sems1, add=True)
            return carry

        lax.fori_loop(0, nb // 2, body, 0)
        if nb % 2 == 1:
            bl = nb - 1
            pltpu.make_async_copy(z_hbm.at[j].at[idxs.at[bl]], rows0,
                                  semg0).wait()
            pltpu.make_async_copy(rows1, acc.at[idxd.at[0]], sems1).wait()
            pltpu.async_copy(rows0, acc.at[idxd.at[bl]], sems0, add=True)
            pltpu.make_async_copy(rows0, acc.at[idxd.at[0]], sems0).wait()
        else:
            pltpu.make_async_copy(rows0, acc.at[idxd.at[0]], sems0).wait()
            pltpu.make_async_copy(rows1, acc.at[idxd.at[0]], sems1).wait()
        plsc.subcore_barrier()
        nfc = ROWS_PT // FC
        bufs = [rows0, rows1]
        gsems = [semg0, semg1]
        osems = [sems0, sems1]
        cds = [None] * nfc
        ods = [None] * nfc
        cds[0] = pltpu.async_copy(acc.at[pl.ds(r0, FC)],
                                  rows0.at[pl.ds(0, FC)], semg0)
        cds[1] = pltpu.async_copy(acc.at[pl.ds(r0 + FC, FC)],
                                  rows1.at[pl.ds(0, FC)], semg1)
        for m in range(nfc):
            buf = bufs[m % 2]
            cds[m].wait()

            def srow(r, carry, m=m, buf=buf):
                idxv = jnp.full((16,), m * FC + r, jnp.int32)
                sc = plsc.load_gather(scale_v, [idxv])
                for jj in range(d2 // 16):
                    vec = buf[r, pl.ds(jj * 16, 16)] * sc
                    if leaky:
                        vec = jnp.where(vec >= 0, vec, 0.01 * vec)
                    buf[r, pl.ds(jj * 16, 16)] = vec
                return carry

            lax.fori_loop(0, FC, srow, 0)
            ods[m] = pltpu.async_copy(buf.at[pl.ds(0, FC)],
                                      out_hbm.at[j, pl.ds(r0 + m * FC, FC)],
                                      osems[m % 2])
            if m + 2 < nfc:
                ods[m].wait()
                cds[m + 2] = pltpu.async_copy(
                    acc.at[pl.ds(r0 + (m + 2) * FC, FC)],
                    bufs[m % 2].at[pl.ds(0, FC)], gsems[m % 2])
        ods[nfc - 2].wait()
        ods[nfc - 1].wait()


@functools.cache
def _prop_f_call(d2, ngroups, leaky):
    kb = 1000 if d2 == 32 else 400
    nb = EPT // kb
    mesh = plsc.VectorSubcoreMesh(core_axis_name="c", subcore_axis_name="s")
    return pl.kernel(
        functools.partial(_prop_f_body, ngroups, d2, kb, leaky),
        out_type=jax.ShapeDtypeStruct((ngroups * NC, NNP, d2), jnp.float32),
        mesh=mesh,
        compiler_params=pltpu.CompilerParams(use_tc_tiling_on_sc=False,
                                             needs_layout_passes=False),
        scratch_types=[
            pltpu.VMEM((nb, kb), jnp.int32),
            pltpu.VMEM((nb, kb), jnp.int32),
            pltpu.VMEM((kb, d2), jnp.float32),
            pltpu.VMEM((kb, d2), jnp.float32),
            pltpu.VMEM((ROWS_PT,), jnp.float32),
            pltpu.SemaphoreType.DMA,
            pltpu.SemaphoreType.DMA,
            pltpu.SemaphoreType.DMA,
            pltpu.SemaphoreType.DMA,
            pltpu.VMEM_SHARED((NNP, d2), jnp.float32),
        ],
    )


def _alpha(ek, v):
    logits = jnp.sum(ek * v, axis=1, keepdims=True)
    m = jnp.max(logits)
    e = jnp.exp(logits - m)
    return e / jnp.sum(e)


def _mm_body(pspec, dout, d2, *refs):
    nch = dout // d2
    nparts = len(pspec)
    parts = refs[:nparts]
    ws = refs[nparts:2 * nparts]
    dinv_r, ek_r, v_r, b_r = refs[2 * nparts:2 * nparts + 4]
    z0m_r, z1m_r, s2_r = refs[2 * nparts + 4:]
    acc = None
    for p, w, kind in zip(parts, ws, pspec):
        pv = p[...]
        if kind[0] == '3d':
            pv = pv[0]
        d = jnp.dot(pv, w[...], preferred_element_type=jnp.float32)
        acc = d if acc is None else acc + d
    al = _alpha(ek_r[...], v_r[...])
    dv = dinv_r[...]
    idv = 1.0 / dv
    z0m = acc[:, :dout] * (al[0:1, :] * idv) + idv * b_r[...]
    z1m = acc[:, dout:2 * dout] * (al[1:2, :] * idv)
    s2 = acc[:, 2 * dout:] * (al[2:3, :] * dv)
    for j in range(nch):
        z0m_r[j] = z0m[:, j * d2:(j + 1) * d2]
        z1m_r[j] = z1m[:, j * d2:(j + 1) * d2]
        s2_r[j] = s2[:, j * d2:(j + 1) * d2]


@functools.cache
def _mm_call(pspec, dout, d2):
    nch = dout // d2
    in_specs = []
    for kind in pspec:
        if kind[0] == '2d':
            in_specs.append(pl.BlockSpec((_BM, kind[1]), lambda i: (i, 0)))
        else:
            jj = kind[2]
            in_specs.append(pl.BlockSpec((1, _BM, kind[1]),
                                         lambda i, jj=jj: (jj, i, 0)))
    for kind in pspec:
        in_specs.append(pl.BlockSpec((kind[1], 3 * dout), lambda i: (0, 0)))
    in_specs += [pl.BlockSpec((_BM, 1), lambda i: (i, 0)),
                 pl.BlockSpec((3, 16), lambda i: (0, 0)),
                 pl.BlockSpec((1, 16), lambda i: (0, 0)),
                 pl.BlockSpec((1, dout), lambda i: (0, 0))]
    spec_np = pl.BlockSpec((nch, _BM, d2), lambda i: (0, i, 0))
    out_specs = (spec_np, spec_np, spec_np)
    out_shape = (
        jax.ShapeDtypeStruct((nch, NNP, d2), jnp.float32),
        jax.ShapeDtypeStruct((nch, NNP, d2), jnp.float32),
        jax.ShapeDtypeStruct((nch, NN, d2), jnp.float32),
    )
    return pl.pallas_call(
        functools.partial(_mm_body, pspec, dout, d2),
        grid=(_GRID,),
        in_specs=in_specs,
        out_specs=out_specs,
        out_shape=out_shape,
    )


def _dinv_body(da_r, dinv_r, dvsq_r):
    da = da_r[...]
    deg = da[:NNP, 0:1] + da[NNP:, 0:1]
    dv = lax.rsqrt(jnp.maximum(deg, 1.0))
    dinv_r[...] = dv
    dvsq_r[...] = dv * dv


@functools.cache
def _dinv_call():
    return pl.pallas_call(
        _dinv_body,
        out_shape=(jax.ShapeDtypeStruct((NNP, 1), jnp.float32),
                   jax.ShapeDtypeStruct((NNP, 1), jnp.float32)),
    )


def _layer(parts, wcat, b, ek, v, srcs, dinv, dv1, dv2, dout, d2):
    nch = dout // d2
    ngroups = nch // NC
    kb = 1000 if d2 == 32 else 400
    pspec = []
    ops = []
    wparts = []
    off = 0
    for f in parts:
        if f.ndim == 2:
            dp = f.shape[1]
            pspec.append(('2d', dp))
            ops.append(f)
            wparts.append(wcat[off:off + dp])
            off += dp
        else:
            nchp, _, d2p = f.shape
            for j in range(nchp):
                pspec.append(('3d', d2p, j))
                ops.append(f)
                wparts.append(wcat[off:off + d2p])
                off += d2p
    v2 = v.reshape(1, -1)
    z0m, z1m, s2 = _mm_call(tuple(pspec), dout, d2)(
        *ops, *wparts, dinv, ek, v2, b.reshape(1, -1))
    src2d, dst2d = srcs[kb]
    s1c = _prop_f_call(d2, ngroups, False)(src2d, dst2d, s2, z1m, dv2)
    h = _prop_f_call(d2, ngroups, True)(src2d, dst2d, s1c, z0m, dv1)
    return h


def kernel(x, edge_index, W1, b1, Ek1, v1, W2, b2, Ek2, v2,
           W3, b3, Ek3, v3, W4, b4, Ek4, v4):
    src = edge_index[0]
    dst = edge_index[1]
    srcs = {
        400: (src.reshape(NS * 25, 400), dst.reshape(NS * 25, 400)),
        1000: (src.reshape(NS * 10, 1000), dst.reshape(NS * 10, 1000)),
    }

    ones = jnp.ones((KB_D, 16), jnp.float32)
    zeros16 = jnp.zeros((NNP, 16), jnp.float32)
    deg_acc = _deg_call()(dst.reshape(NS * 50, KB_D), ones, zeros16)
    dinv, dvsq = _dinv_call()(deg_acc)
    dv1 = dinv.reshape(NNP)
    dv2 = dvsq.reshape(NNP)

    W4p = jnp.pad(W4, ((0, 0), (0, 0), (0, 24)))
    b4p = jnp.pad(b4, (0, 24))

    def wcat(W):
        return jnp.concatenate([W[0], W[1], W[2]], axis=1)

    h1 = _layer([x], wcat(W1), b1, Ek1, v1, srcs, dinv, dv1, dv2, 256, 64)
    h2 = _layer([h1, x], wcat(W2), b2, Ek2, v2, srcs, dinv, dv1, dv2, 128, 64)
    h3 = _layer([h2, h1, x], wcat(W3), b3, Ek3, v3, srcs, dinv, dv1, dv2,
                64, 32)
    h4 = _layer([h3, h2, h1, x], wcat(W4p), b4p, Ek4, v4, srcs, dinv, dv1, dv2,
                64, 32)
    return jnp.concatenate([h4[0, :NN], h4[1, :NN]], axis=1)[:, :40]


# speedup vs baseline: 1.1082x; 1.0006x over previous
import functools

import jax
import jax.numpy as jnp
from jax import lax
from jax.experimental import pallas as pl
from jax.experimental.pallas import tpu as pltpu
from jax.experimental.pallas import tpu_sc as plsc

NN = 10000
NNP = 10240
EE = 160000
NC = 2
NS = 16
EPT = EE // NS
ROWS_PT = NNP // NS
KB_D = 200
DEG_EPT = EE // (NC * NS)
NB_D = DEG_EPT // KB_D
FC = 320

_BM = 400
_GRID = NN // _BM


def _deg_body(dst_hbm, ones_hbm, zeros_hbm, out_hbm, idxd, ones_v, sems, acc):
    c = lax.axis_index("c")
    s = lax.axis_index("s")
    r0 = s * ROWS_PT
    t = c * NS + s
    pltpu.sync_copy(zeros_hbm.at[pl.ds(r0, ROWS_PT)], acc.at[pl.ds(r0, ROWS_PT)])
    pltpu.sync_copy(dst_hbm.at[pl.ds(t * NB_D, NB_D)], idxd)
    pltpu.sync_copy(ones_hbm, ones_v)
    plsc.subcore_barrier()

    def fire(b, carry):
        pltpu.async_copy(ones_v, acc.at[idxd.at[b]], sems, add=True)
        return carry

    lax.fori_loop(0, NB_D, fire, 0)

    def drain(b, carry):
        pltpu.make_async_copy(ones_v, acc.at[idxd.at[0]], sems).wait()
        return carry

    lax.fori_loop(0, NB_D, drain, 0)
    plsc.subcore_barrier()
    pltpu.sync_copy(acc.at[pl.ds(r0, ROWS_PT)],
                    out_hbm.at[pl.ds(c * NNP + r0, ROWS_PT)])


@functools.cache
def _deg_call():
    mesh = plsc.VectorSubcoreMesh(core_axis_name="c", subcore_axis_name="s")
    return pl.kernel(
        _deg_body,
        out_type=jax.ShapeDtypeStruct((NC * NNP, 16), jnp.float32),
        mesh=mesh,
        compiler_params=pltpu.CompilerParams(use_tc_tiling_on_sc=False),
        scratch_types=[
            pltpu.VMEM((NB_D, KB_D), jnp.int32),
            pltpu.VMEM((KB_D, 16), jnp.float32),
            pltpu.SemaphoreType.DMA,
            pltpu.VMEM_SHARED((NNP, 16), jnp.float32),
        ],
    )


def _prop_f_body(ngroups, d2, kb, leaky,
                 src_hbm, dst_hbm, z_hbm, init_hbm, scale_hbm, out_hbm,
                 idxs, idxd, rows0, rows1, scale_v,
                 semg0, semg1, sems0, sems1, acc):
    nb = EPT // kb
    c = lax.axis_index("c")
    s = lax.axis_index("s")
    r0 = s * ROWS_PT
    pd1 = pltpu.async_copy(dst_hbm.at[pl.ds(s * nb, nb)], idxd, semg0)
    pd2 = pltpu.async_copy(src_hbm.at[pl.ds(s * nb, nb)], idxs, semg1)
    pd3 = pltpu.async_copy(scale_hbm.at[pl.ds(r0, ROWS_PT)], scale_v, sems0)
    pd4 = pltpu.async_copy(init_hbm.at[c, pl.ds(r0, ROWS_PT)],
                           acc.at[pl.ds(r0, ROWS_PT)], sems1)
    pd1.wait()
    pd2.wait()
    pd3.wait()
    pd4.wait()
    for g in range(ngroups):
        j = g * NC + c
        if g > 0:
            pltpu.sync_copy(init_hbm.at[j, pl.ds(r0, ROWS_PT)],
                            acc.at[pl.ds(r0, ROWS_PT)])
        plsc.subcore_barrier()
        pltpu.async_copy(z_hbm.at[j].at[idxs.at[0]], rows0, semg0)

        def body(i, carry):
            b0 = 2 * i
            b1 = 2 * i + 1
            pltpu.make_async_copy(z_hbm.at[j].at[idxs.at[b0]], rows0, semg0).wait()

            @pl.when(i > 0)
            def _():
                pltpu.make_async_copy(rows1, acc.at[idxd.at[0]], sems1).wait()

            pltpu.async_copy(z_hbm.at[j].at[idxs.at[b1]], rows1, semg1)
            pltpu.async_copy(rows0, acc.at[idxd.at[b0]], sems0, add=True)
            pltpu.make_async_copy(z_hbm.at[j].at[idxs.at[b1]], rows1, semg1).wait()

            lim = nb // 2 - 1 if nb % 2 == 0 else nb // 2

            @pl.when(i < lim)
            def _():
                pltpu.make_async_copy(rows0, acc.at[idxd.at[0]], sems0).wait()
                pltpu.async_copy(z_hbm.at[j].at[idxs.at[b0 + 2]], rows0, semg0)

            pltpu.async_copy(rows1, acc.at[idxd.at[b1]], sems1, add=True)
            return carry

        lax.fori_loop(0, nb // 2, body, 0)
        if nb % 2 == 1:
            bl = nb - 1
            pltpu.make_async_copy(z_hbm.at[j].at[idxs.at[bl]], rows0,
                                  semg0).wait()
            pltpu.make_async_copy(rows1, acc.at[idxd.at[0]], sems1).wait()
            pltpu.async_copy(rows0, acc.at[idxd.at[bl]], sems0, add=True)
            pltpu.make_async_copy(rows0, acc.at[idxd.at[0]], sems0).wait()
        else:
            pltpu.make_async_copy(rows0, acc.at[idxd.at[0]], sems0).wait()
            pltpu.make_async_copy(rows1, acc.at[idxd.at[0]], sems1).wait()
        plsc.subcore_barrier()
        nfc = ROWS_PT // FC
        bufs = [rows0, rows1]
        gsems = [semg0, semg1]
        osems = [sems0, sems1]
        cds = [None] * nfc
        ods = [None] * nfc
        cds[0] = pltpu.async_copy(acc.at[pl.ds(r0, FC)],
                                  rows0.at[pl.ds(0, FC)], semg0)
        cds[1] = pltpu.async_copy(acc.at[pl.ds(r0 + FC, FC)],
                                  rows1.at[pl.ds(0, FC)], semg1)
        for m in range(nfc):
            buf = bufs[m % 2]
            cds[m].wait()

            def srow(r, carry, m=m, buf=buf):
                idxv = jnp.full((16,), m * FC + r, jnp.int32)
                sc = plsc.load_gather(scale_v, [idxv])
                for jj in range(d2 // 16):
                    vec = buf[r, pl.ds(jj * 16, 16)] * sc
                    if leaky:
                        vec = jnp.where(vec >= 0, vec, 0.01 * vec)
                    buf[r, pl.ds(jj * 16, 16)] = vec
                return carry

            lax.fori_loop(0, FC, srow, 0)
            ods[m] = pltpu.async_copy(buf.at[pl.ds(0, FC)],
                                      out_hbm.at[j, pl.ds(r0 + m * FC, FC)],
                                      osems[m % 2])
            if m + 2 < nfc:
                ods[m].wait()
                cds[m + 2] = pltpu.async_copy(
                    acc.at[pl.ds(r0 + (m + 2) * FC, FC)],
                    bufs[m % 2].at[pl.ds(0, FC)], gsems[m % 2])
        ods[nfc - 2].wait()
        ods[nfc - 1].wait()


@functools.cache
def _prop_f_call(d2, ngroups, leaky):
    kb = 1000 if d2 == 32 else 400
    nb = EPT // kb
    mesh = plsc.VectorSubcoreMesh(core_axis_name="c", subcore_axis_name="s")
    return pl.kernel(
        functools.partial(_prop_f_body, ngroups, d2, kb, leaky),
        out_type=jax.ShapeDtypeStruct((ngroups * NC, NNP, d2), jnp.float32),
        mesh=mesh,
        compiler_params=pltpu.CompilerParams(use_tc_tiling_on_sc=False,
                                             needs_layout_passes=False),
        scratch_types=[
            pltpu.VMEM((nb, kb), jnp.int32),
            pltpu.VMEM((nb, kb), jnp.int32),
            pltpu.VMEM((kb, d2), jnp.float32),
            pltpu.VMEM((kb, d2), jnp.float32),
            pltpu.VMEM((ROWS_PT,), jnp.float32),
            pltpu.SemaphoreType.DMA,
            pltpu.SemaphoreType.DMA,
            pltpu.SemaphoreType.DMA,
            pltpu.SemaphoreType.DMA,
            pltpu.VMEM_SHARED((NNP, d2), jnp.float32),
        ],
    )


def _alpha(ek, v):
    logits = jnp.sum(ek * v, axis=1, keepdims=True)
    m = jnp.max(logits)
    e = jnp.exp(logits - m)
    return e / jnp.sum(e)


def _mm_body(pspec, dout, d2, *refs):
    nch = dout // d2
    nparts = len(pspec)
    parts = refs[:nparts]
    ws = refs[nparts:2 * nparts]
    dinv_r, ek_r, v_r, b_r = refs[2 * nparts:2 * nparts + 4]
    z0m_r, z1m_r, s2_r = refs[2 * nparts + 4:]
    acc = None
    for p, w, kind in zip(parts, ws, pspec):
        pv = p[...]
        if kind[0] == '3d':
            pv = pv[0]
        d = jnp.dot(pv, w[...], preferred_element_type=jnp.float32)
        acc = d if acc is None else acc + d
    al = _alpha(ek_r[...], v_r[...])
    dv = dinv_r[...]
    idv = 1.0 / dv
    z0m = acc[:, :dout] * (al[0:1, :] * idv) + idv * b_r[...]
    z1m = acc[:, dout:2 * dout] * (al[1:2, :] * idv)
    s2 = acc[:, 2 * dout:] * (al[2:3, :] * dv)
    for j in range(nch):
        z0m_r[j] = z0m[:, j * d2:(j + 1) * d2]
        z1m_r[j] = z1m[:, j * d2:(j + 1) * d2]
        s2_r[j] = s2[:, j * d2:(j + 1) * d2]


@functools.cache
def _mm_call(pspec, dout, d2):
    nch = dout // d2
    in_specs = []
    for kind in pspec:
        if kind[0] == '2d':
            in_specs.append(pl.BlockSpec((_BM, kind[1]), lambda i: (i, 0)))
        else:
            jj = kind[2]
            in_specs.append(pl.BlockSpec((1, _BM, kind[1]),
                                         lambda i, jj=jj: (jj, i, 0)))
    for kind in pspec:
        in_specs.append(pl.BlockSpec((kind[1], 3 * dout), lambda i: (0, 0)))
    in_specs += [pl.BlockSpec((_BM, 1), lambda i: (i, 0)),
                 pl.BlockSpec((3, 16), lambda i: (0, 0)),
                 pl.BlockSpec((1, 16), lambda i: (0, 0)),
                 pl.BlockSpec((1, dout), lambda i: (0, 0))]
    spec_np = pl.BlockSpec((nch, _BM, d2), lambda i: (0, i, 0))
    out_specs = (spec_np, spec_np, spec_np)
    out_shape = (
        jax.ShapeDtypeStruct((nch, NNP, d2), jnp.float32),
        jax.ShapeDtypeStruct((nch, NNP, d2), jnp.float32),
        jax.ShapeDtypeStruct((nch, NN, d2), jnp.float32),
    )
    return pl.pallas_call(
        functools.partial(_mm_body, pspec, dout, d2),
        grid=(_GRID,),
        in_specs=in_specs,
        out_specs=out_specs,
        out_shape=out_shape,
    )


def _dinv_body(da_r, dinv_r, dvsq_r):
    da = da_r[...]
    deg = da[:NNP, 0:1] + da[NNP:, 0:1]
    dv = lax.rsqrt(jnp.maximum(deg, 1.0))
    dinv_r[...] = dv
    dvsq_r[...] = dv * dv


@functools.cache
def _dinv_call():
    return pl.pallas_call(
        _dinv_body,
        out_shape=(jax.ShapeDtypeStruct((NNP, 1), jnp.float32),
                   jax.ShapeDtypeStruct((NNP, 1), jnp.float32)),
    )


def _layer(parts, wcat, b, ek, v, srcs, dinv, dv1, dv2, dout, d2):
    nch = dout // d2
    ngroups = nch // NC
    kb = 1000 if d2 == 32 else 400
    pspec = []
    ops = []
    wparts = []
    off = 0
    for f in parts:
        if f.ndim == 2:
            dp = f.shape[1]
            pspec.append(('2d', dp))
            ops.append(f)
            wparts.append(wcat[off:off + dp])
            off += dp
        else:
            nchp, _, d2p = f.shape
            for j in range(nchp):
                pspec.append(('3d', d2p, j))
                ops.append(f)
                wparts.append(wcat[off:off + d2p])
                off += d2p
    v2 = v.reshape(1, -1)
    z0m, z1m, s2 = _mm_call(tuple(pspec), dout, d2)(
        *ops, *wparts, dinv, ek, v2, b.reshape(1, -1))
    src2d, dst2d = srcs[kb]
    s1c = _prop_f_call(d2, ngroups, False)(src2d, dst2d, s2, z1m, dv2)
    h = _prop_f_call(d2, ngroups, True)(src2d, dst2d, s1c, z0m, dv1)
    return h


def kernel(x, edge_index, W1, b1, Ek1, v1, W2, b2, Ek2, v2,
           W3, b3, Ek3, v3, W4, b4, Ek4, v4):
    src = edge_index[0]
    dst = edge_index[1]
    srcs = {
        400: (src.reshape(NS * 25, 400), dst.reshape(NS * 25, 400)),
        1000: (src.reshape(NS * 10, 1000), dst.reshape(NS * 10, 1000)),
    }

    ones = jnp.ones((KB_D, 16), jnp.float32)
    zeros16 = jnp.zeros((NNP, 16), jnp.float32)
    deg_acc = _deg_call()(dst.reshape(NS * 50, KB_D), ones, zeros16)
    dinv, dvsq = _dinv_call()(deg_acc)
    dv1 = dinv.reshape(NNP)
    dv2 = dvsq.reshape(NNP)

    W4p = jnp.pad(W4, ((0, 0), (0, 0), (0, 24)))
    b4p = jnp.pad(b4, (0, 24))

    def wcat(W):
        return jnp.concatenate([W[0], W[1], W[2]], axis=1)

    h1 = _layer([x], wcat(W1), b1, Ek1, v1, srcs, dinv, dv1, dv2, 256, 64)
    h2 = _layer([h1, x], wcat(W2), b2, Ek2, v2, srcs, dinv, dv1, dv2, 128, 64)
    h3 = _layer([h2, h1, x], wcat(W3), b3, Ek3, v3, srcs, dinv, dv1, dv2,
                64, 32)
    h4 = _layer([h3, h2, h1, x], wcat(W4p), b4p, Ek4, v4, srcs, dinv, dv1, dv2,
                64, 32)
    return jnp.concatenate([h4[0, :NN], h4[1, :NN]], axis=1)[:, :40]
